# Initial kernel scaffold; baseline (speedup 1.0000x reference)
import functools

import jax
import jax.numpy as jnp
from jax import lax
from jax.experimental import pallas as pl
from jax.experimental.pallas import tpu as pltpu
from jax.experimental.pallas import tpu_sc as plsc

_D = 32
_G = 128
_K = 8
_NC = 2
_NS = 16
_NW = _NC * _NS


@functools.lru_cache(maxsize=None)
def _build(B):
    R = B // _G
    J = R // _NW
    NB = J // _K
    mesh = plsc.VectorSubcoreMesh(core_axis_name="c", subcore_axis_name="s")

    @functools.partial(
        pl.kernel,
        mesh=mesh,
        out_type=jax.ShapeDtypeStruct((R, _G, _D), jnp.float32),
        scratch_types=[
            pltpu.VMEM((2, _K, _G), jnp.int32),
            pltpu.VMEM((2, _K, _G, _D), jnp.float32),
            pltpu.SemaphoreType.DMA,
            pltpu.SemaphoreType.DMA,
            pltpu.SemaphoreType.DMA,
            pltpu.SemaphoreType.DMA,
        ],
    )
    def emb(idx_hbm, table_hbm, out_hbm, idx_v, rows_v, sg0, sg1, so0, so1):
        wid = lax.axis_index("s") * _NC + lax.axis_index("c")
        row0 = wid * J
        sg = (sg0, sg1)
        so = (so0, so1)

        def stage_and_fire(p, h):
            r = row0 + p * _K
            pltpu.sync_copy(idx_hbm.at[pl.ds(r, _K)], idx_v.at[h])
            for j in range(_K):
                pltpu.async_copy(table_hbm.at[idx_v.at[h, j]], rows_v.at[h, j], sg[h])

        def drain_gathers(h):
            pltpu.make_async_copy(out_hbm.at[pl.ds(0, _K)], rows_v.at[h], sg[h]).wait()

        def fire_out(p, h):
            r = row0 + p * _K
            pltpu.async_copy(rows_v.at[h], out_hbm.at[pl.ds(r, _K)], so[h])

        def drain_out(h):
            pltpu.make_async_copy(rows_v.at[h], out_hbm.at[pl.ds(0, _K)], so[h]).wait()

        stage_and_fire(0, 0)

        def body(i, carry):
            p0 = i * 2
            for h in (0, 1):
                p = p0 + h
                hn = 1 - h

                @pl.when(p + 1 < NB)
                def _fire_next():
                    @pl.when(p >= 1)
                    def _free_half():
                        drain_out(hn)

                    stage_and_fire(p + 1, hn)

                drain_gathers(h)
                fire_out(p, h)
            return carry

        lax.fori_loop(0, NB // 2, body, 0)
        drain_out(0)
        drain_out(1)

    return emb


def kernel(inputs, table):
    batch, hist = inputs.shape
    B = batch * hist
    idx2 = inputs.reshape(B // _G, _G)
    out = _build(B)(idx2, table)
    return out.reshape(batch, hist, _D)


# speedup vs baseline: 4.9513x; 4.9513x over previous
import functools

import jax
import jax.numpy as jnp
from jax import lax
from jax.experimental import pallas as pl
from jax.experimental.pallas import tpu as pltpu
from jax.experimental.pallas import tpu_sc as plsc

_D = 32
_G = 128
_K = 8
_NC = 2
_NS = 16
_NW = _NC * _NS


@functools.lru_cache(maxsize=None)
def _build(B):
    R = B // _G
    J = R // _NW
    NB = J // _K
    mesh = plsc.VectorSubcoreMesh(core_axis_name="c", subcore_axis_name="s")

    @functools.partial(
        pl.kernel,
        mesh=mesh,
        compiler_params=pltpu.CompilerParams(use_tc_tiling_on_sc=False),
        out_type=jax.ShapeDtypeStruct((R, _G, _D), jnp.float32),
        scratch_types=[
            pltpu.VMEM((2, _K, _G), jnp.int32),
            pltpu.VMEM((2, _K, _G, _D), jnp.float32),
            pltpu.SemaphoreType.DMA,
            pltpu.SemaphoreType.DMA,
            pltpu.SemaphoreType.DMA,
            pltpu.SemaphoreType.DMA,
        ],
    )
    def emb(idx_hbm, table_hbm, out_hbm, idx_v, rows_v, sg0, sg1, so0, so1):
        wid = lax.axis_index("s") * _NC + lax.axis_index("c")
        row0 = wid * J
        sg = (sg0, sg1)
        so = (so0, so1)

        def stage_and_fire(p, h):
            r = row0 + p * _K
            pltpu.sync_copy(idx_hbm.at[pl.ds(r, _K)], idx_v.at[h])
            for j in range(_K):
                pltpu.async_copy(table_hbm.at[idx_v.at[h, j]], rows_v.at[h, j], sg[h])

        def drain_gathers(h):
            pltpu.make_async_copy(out_hbm.at[pl.ds(0, _K)], rows_v.at[h], sg[h]).wait()

        def fire_out(p, h):
            r = row0 + p * _K
            pltpu.async_copy(rows_v.at[h], out_hbm.at[pl.ds(r, _K)], so[h])

        def drain_out(h):
            pltpu.make_async_copy(rows_v.at[h], out_hbm.at[pl.ds(0, _K)], so[h]).wait()

        stage_and_fire(0, 0)

        def body(i, carry):
            p0 = i * 2
            for h in (0, 1):
                p = p0 + h
                hn = 1 - h

                @pl.when(p + 1 < NB)
                def _fire_next():
                    @pl.when(p >= 1)
                    def _free_half():
                        drain_out(hn)

                    stage_and_fire(p + 1, hn)

                drain_gathers(h)
                fire_out(p, h)
            return carry

        lax.fori_loop(0, NB // 2, body, 0)
        drain_out(0)
        drain_out(1)

    return emb


def kernel(inputs, table):
    batch, hist = inputs.shape
    B = batch * hist
    idx2 = inputs.reshape(B // _G, _G)
    out = _build(B)(idx2, table)
    return out.reshape(batch, hist, _D)
